# baseline (speedup 1.0000x reference)
import functools

import jax
import jax.numpy as jnp
from jax import lax
from jax.experimental import pallas as pl
from jax.experimental.pallas import tpu as pltpu
from jax.experimental.pallas import tpu_sc as plsc

B = 16384
D = 64
NC = 2
NS = 16
NW = NC * NS
BPW = B // NW
CH = 128
NCH = BPW // CH
L = 16


def _body(uidx_hbm, iidx_hbm, utab_hbm, itab_hbm, out_hbm,
          uidx_v, iidx_v, urows_v, irows_v, out_v, sem):
    wid = lax.axis_index("s") * NC + lax.axis_index("c")

    pltpu.sync_copy(uidx_hbm.at[wid], uidx_v)
    pltpu.sync_copy(iidx_hbm.at[wid], iidx_v)

    copies = []
    for j in range(NCH):
        dst = urows_v.at[pl.ds(j * CH, CH), :]
        copies.append(pltpu.async_copy(utab_hbm.at[uidx_v.at[j]], dst, sem))
    for j in range(NCH):
        dst = irows_v.at[pl.ds(j * CH, CH), :]
        copies.append(pltpu.async_copy(itab_hbm.at[iidx_v.at[j]], dst, sem))
    for c in copies:
        c.wait()

    lane = lax.iota(jnp.int32, L)

    def lane_sum(x):
        for k in (8, 4, 2, 1):
            x = x + x.at[lane ^ k].get(mode="promise_in_bounds", unique_indices=True)
        return x

    @pl.loop(0, BPW // L)
    def _group(g):
        base = g * L
        res = jnp.zeros((L,), jnp.float32)
        for l in range(L):
            i = base + l
            acc = urows_v[i, pl.ds(0, L)] * irows_v[i, pl.ds(0, L)]
            for k in range(1, D // L):
                acc += urows_v[i, pl.ds(k * L, L)] * irows_v[i, pl.ds(k * L, L)]
            res = jnp.where(lane == l, lane_sum(acc), res)
        out_v[pl.ds(base, L)] = res

    pltpu.sync_copy(out_v, out_hbm.at[wid])


@functools.partial(
    pl.kernel,
    out_type=jax.ShapeDtypeStruct((NW, BPW), jnp.float32),
    mesh=plsc.VectorSubcoreMesh(core_axis_name="c", subcore_axis_name="s"),
    compiler_params=pltpu.CompilerParams(use_tc_tiling_on_sc=False),
    scratch_types=[
        pltpu.VMEM((NCH, CH), jnp.int32),
        pltpu.VMEM((NCH, CH), jnp.int32),
        pltpu.VMEM((BPW, D), jnp.float32),
        pltpu.VMEM((BPW, D), jnp.float32),
        pltpu.VMEM((BPW,), jnp.float32),
        pltpu.SemaphoreType.DMA,
    ],
)
def _two_tower(*args):
    _body(*args)


def kernel(user_idx, item_idx, user_table, item_table):
    uidx = jnp.asarray(user_idx, jnp.int32).reshape(NW, NCH, CH)
    iidx = jnp.asarray(item_idx, jnp.int32).reshape(NW, NCH, CH)
    out = _two_tower(uidx, iidx, user_table, item_table)
    return out.reshape(B)


# speedup vs baseline: 1.5515x; 1.5515x over previous
import functools

import jax
import jax.numpy as jnp
from jax import lax
from jax.experimental import pallas as pl
from jax.experimental.pallas import tpu as pltpu
from jax.experimental.pallas import tpu_sc as plsc

B = 16384
D = 64
NC = 2
NS = 16
NW = NC * NS
BPW = B // NW
CH = 128
NCH = BPW // CH
L = 16


def _body(uidx_hbm, iidx_hbm, utab_hbm, itab_hbm, out_hbm,
          uidx_v, iidx_v, urows_v, irows_v, out_v, sem):
    wid = lax.axis_index("s") * NC + lax.axis_index("c")

    pltpu.sync_copy(uidx_hbm.at[wid], uidx_v)
    pltpu.sync_copy(iidx_hbm.at[wid], iidx_v)

    lane = lax.iota(jnp.int32, L)

    def lane_sum(x):
        for k in (8, 4, 2, 1):
            x = x + x.at[lane ^ k].get(mode="promise_in_bounds", unique_indices=True)
        return x

    def fire(c):
        base = c * CH

        @pl.loop(0, CH // L)
        def _fire(g):
            uvec = uidx_v[pl.ds(base + g * L, L)]
            ivec = iidx_v[pl.ds(base + g * L, L)]
            for l in range(L):
                dst = g * L + l
                pltpu.async_copy(utab_hbm.at[uvec[l]], urows_v.at[dst], sem)
                pltpu.async_copy(itab_hbm.at[ivec[l]], irows_v.at[dst], sem)

    def drain_and_compute(c):
        base = c * CH

        @pl.loop(0, CH, unroll=8)
        def _drain(i):
            pltpu.make_async_copy(utab_hbm.at[0], urows_v.at[0], sem).wait()
            pltpu.make_async_copy(itab_hbm.at[0], irows_v.at[0], sem).wait()

        @pl.loop(0, CH // L)
        def _group(g):
            gbase = g * L
            res = jnp.zeros((L,), jnp.float32)
            for l in range(L):
                i = gbase + l
                acc = urows_v[i, pl.ds(0, L)] * irows_v[i, pl.ds(0, L)]
                for k in range(1, D // L):
                    acc += urows_v[i, pl.ds(k * L, L)] * irows_v[i, pl.ds(k * L, L)]
                res = jnp.where(lane == l, lane_sum(acc), res)
            out_v[pl.ds(base + gbase, L)] = res

    for c in range(NCH):
        fire(c)
        drain_and_compute(c)

    pltpu.sync_copy(out_v, out_hbm.at[wid])


@functools.partial(
    pl.kernel,
    out_type=jax.ShapeDtypeStruct((NW, BPW), jnp.float32),
    mesh=plsc.VectorSubcoreMesh(core_axis_name="c", subcore_axis_name="s"),
    scratch_types=[
        pltpu.VMEM((BPW,), jnp.int32),
        pltpu.VMEM((BPW,), jnp.int32),
        pltpu.VMEM((CH, D), jnp.float32),
        pltpu.VMEM((CH, D), jnp.float32),
        pltpu.VMEM((BPW,), jnp.float32),
        pltpu.SemaphoreType.DMA,
    ],
)
def _two_tower(*args):
    _body(*args)


def kernel(user_idx, item_idx, user_table, item_table):
    uidx = jnp.asarray(user_idx, jnp.int32).reshape(NW, BPW)
    iidx = jnp.asarray(item_idx, jnp.int32).reshape(NW, BPW)
    out = _two_tower(uidx, iidx, user_table, item_table)
    return out.reshape(B)


# speedup vs baseline: 1.5529x; 1.0009x over previous
import functools

import jax
import jax.numpy as jnp
from jax import lax
from jax.experimental import pallas as pl
from jax.experimental.pallas import tpu as pltpu
from jax.experimental.pallas import tpu_sc as plsc

B = 16384
D = 64
NC = 2
NS = 16
NW = NC * NS
BPW = B // NW
CH = 128
NCH = BPW // CH
L = 16
NSEM = 8


def _body(uidx_hbm, iidx_hbm, utab_hbm, itab_hbm, out_hbm,
          uidx_v, iidx_v, urows_v, irows_v, out_v, sems):
    wid = lax.axis_index("s") * NC + lax.axis_index("c")

    pltpu.sync_copy(uidx_hbm.at[wid], uidx_v)
    pltpu.sync_copy(iidx_hbm.at[wid], iidx_v)

    lane = lax.iota(jnp.int32, L)

    def lane_sum(x):
        for k in (8, 4, 2, 1):
            x = x + x.at[lane ^ k].get(mode="promise_in_bounds", unique_indices=True)
        return x

    def fire(c):
        base = c * CH

        @pl.loop(0, CH // L)
        def _fire(g):
            uvec = uidx_v[pl.ds(base + g * L, L)]
            ivec = iidx_v[pl.ds(base + g * L, L)]
            for l in range(L):
                dst = g * L + l
                pltpu.async_copy(utab_hbm.at[uvec[l]], urows_v.at[dst],
                                 sems.at[(2 * l) % NSEM])
                pltpu.async_copy(itab_hbm.at[ivec[l]], irows_v.at[dst],
                                 sems.at[(2 * l + 1) % NSEM])

    def drain_and_compute(c):
        base = c * CH

        @pl.loop(0, CH // L)
        def _drain(g):
            for l in range(L):
                pltpu.make_async_copy(utab_hbm.at[0], urows_v.at[0],
                                      sems.at[(2 * l) % NSEM]).wait()
                pltpu.make_async_copy(itab_hbm.at[0], irows_v.at[0],
                                      sems.at[(2 * l + 1) % NSEM]).wait()

        @pl.loop(0, CH // L)
        def _group(g):
            gbase = g * L
            res = jnp.zeros((L,), jnp.float32)
            for l in range(L):
                i = gbase + l
                acc = urows_v[i, pl.ds(0, L)] * irows_v[i, pl.ds(0, L)]
                for k in range(1, D // L):
                    acc += urows_v[i, pl.ds(k * L, L)] * irows_v[i, pl.ds(k * L, L)]
                res = jnp.where(lane == l, lane_sum(acc), res)
            out_v[pl.ds(base + gbase, L)] = res

    for c in range(NCH):
        fire(c)
        drain_and_compute(c)

    pltpu.sync_copy(out_v, out_hbm.at[wid])


@functools.partial(
    pl.kernel,
    out_type=jax.ShapeDtypeStruct((NW, BPW), jnp.float32),
    mesh=plsc.VectorSubcoreMesh(core_axis_name="c", subcore_axis_name="s"),
    scratch_types=[
        pltpu.VMEM((BPW,), jnp.int32),
        pltpu.VMEM((BPW,), jnp.int32),
        pltpu.VMEM((CH, D), jnp.float32),
        pltpu.VMEM((CH, D), jnp.float32),
        pltpu.VMEM((BPW,), jnp.float32),
        pltpu.SemaphoreType.DMA((NSEM,)),
    ],
)
def _two_tower(*args):
    _body(*args)


def kernel(user_idx, item_idx, user_table, item_table):
    uidx = jnp.asarray(user_idx, jnp.int32).reshape(NW, BPW)
    iidx = jnp.asarray(item_idx, jnp.int32).reshape(NW, BPW)
    out = _two_tower(uidx, iidx, user_table, item_table)
    return out.reshape(B)
